# baseline (speedup 1.0000x reference)
import functools

import jax
import jax.numpy as jnp
from jax import lax
from jax.experimental import pallas as pl
from jax.experimental.pallas import tpu as pltpu
from jax.experimental.pallas import tpu_sc as plsc

_ROWS, _COLS = 64, 32768
_N = _ROWS * _COLS
_PCT = 0.99999
_K_MIN = int(_N * (1 - _PCT))
_K_MAX = int(_N * _PCT)

_NW = 32
_CHUNK = _N // _NW
_L = 16
_NGROUPS = 32
_GROUP = _CHUNK // _NGROUPS
_CAP = 1024
_NBLK = _CHUNK // (8 * _L)
_CAND = _NW * _CAP
_K_TOP = _CAND - (_N - _K_MAX)


def _sc_filter_body(x_hbm, lo_hbm, hi_hbm, chunk_v, blo_v, bhi_v, summ_v, sem):
    nc = 2
    wid = lax.axis_index("s") * nc + lax.axis_index("c")

    c1 = pltpu.async_copy(x_hbm.at[2 * wid], chunk_v.at[pl.ds(0, _COLS)], sem)
    c2 = pltpu.async_copy(
        x_hbm.at[2 * wid + 1], chunk_v.at[pl.ds(_COLS, _COLS)], sem)

    inf_v = jnp.full((_L,), jnp.inf, jnp.float32)
    ninf_v = jnp.full((_L,), -jnp.inf, jnp.float32)

    def pad_body(i, c):
        base = i * (8 * _L)
        for t in range(8):
            blo_v[pl.ds(base + t * _L, _L)] = inf_v
            bhi_v[pl.ds(base + t * _L, _L)] = ninf_v
        return c

    lax.fori_loop(0, _CAP // (8 * _L), pad_body, 0)

    _GV = 204

    def group_scan(g0, ngroups, carry):
        def body(g, c):
            tl, th = c
            base = (g0 + g) * (_GV * _L)

            def inner(j, mm):
                los = list(mm[:4])
                his = list(mm[4:])
                b2 = base + j * (12 * _L)
                for t in range(12):
                    v = chunk_v[pl.ds(b2 + t * _L, _L)]
                    los[t % 4] = jnp.minimum(los[t % 4], v)
                    his[t % 4] = jnp.maximum(his[t % 4], v)
                return tuple(los) + tuple(his)

            r = lax.fori_loop(0, _GV // 12, inner,
                              (inf_v,) * 4 + (ninf_v,) * 4)
            vlo = jnp.minimum(jnp.minimum(r[0], r[1]),
                              jnp.minimum(r[2], r[3]))
            vhi = jnp.maximum(jnp.maximum(r[4], r[5]),
                              jnp.maximum(r[6], r[7]))
            gmin = vlo[0]
            gmax = vhi[0]
            for l in range(1, _L):
                gmin = jnp.minimum(gmin, vlo[l])
                gmax = jnp.maximum(gmax, vhi[l])
            return jnp.maximum(tl, gmin), jnp.minimum(th, gmax)

        return lax.fori_loop(0, ngroups, body, carry)

    c1.wait()
    carry0 = (jnp.float32(-jnp.inf), jnp.float32(jnp.inf))
    carry0 = group_scan(0, 10, carry0)
    c2.wait()
    thr_lo, thr_hi = group_scan(10, 10, carry0)

    thr_lo_v = jnp.broadcast_to(thr_lo, (_L,))
    thr_hi_v = jnp.broadcast_to(thr_hi, (_L,))

    def or_tree(ms):
        while len(ms) > 1:
            nxt = [ms[j] | ms[j + 1] for j in range(0, len(ms) - 1, 2)]
            if len(ms) % 2:
                nxt.append(ms[-1])
            ms = nxt
        return ms[0]

    one_v = jnp.full((_L,), 1, jnp.int32)
    zero_v = jnp.full((_L,), 0, jnp.int32)

    def pass_a(b, c):
        base = b * (8 * _L)
        vs = [chunk_v[pl.ds(base + t * _L, _L)] for t in range(8)]
        mlos = [v <= thr_lo_v for v in vs]
        mhis = [v >= thr_hi_v for v in vs]
        any_m = or_tree(mlos) | or_tree(mhis)
        summ_v[pl.ds(b * _L, _L)] = jnp.where(any_m, one_v, zero_v)
        return c

    lax.fori_loop(0, _NBLK, pass_a, 0)

    def rescan(b):
        def f(c):
            p_lo, p_hi = c
            base = b * (8 * _L)
            vs = [chunk_v[pl.ds(base + t * _L, _L)] for t in range(8)]
            mlos = [v <= thr_lo_v for v in vs]
            mhis = [v >= thr_hi_v for v in vs]
            clos = [plsc.all_reduce_population_count(m) for m in mlos]
            chis = [plsc.all_reduce_population_count(m) for m in mhis]
            cls = [c_[0] for c_ in clos]
            chs = [c_[0] for c_ in chis]
            for t in range(8):
                plsc.store_compressed(
                    blo_v.at[pl.ds(p_lo, _L)], vs[t], mask=mlos[t])
                plsc.store_compressed(
                    bhi_v.at[pl.ds(p_hi, _L)], vs[t], mask=mhis[t])
                p_lo = jnp.minimum(p_lo + cls[t], _CAP - _L)
                p_hi = jnp.minimum(p_hi + chs[t], _CAP - _L)
            return p_lo, p_hi

        return f

    def keep(c):
        return c

    def pass_b(g, carry):
        cnts = []
        for t in range(8):
            sv = summ_v[pl.ds((g * 8 + t) * _L, _L)]
            cnts.append(plsc.all_reduce_population_count(sv > 0))
        cs = [c_[0] for c_ in cnts]
        for t in range(8):
            carry = lax.cond(cs[t] > 0, rescan(g * 8 + t), keep, carry)
        return carry

    lax.fori_loop(0, _NBLK // 8, pass_b, (jnp.int32(0), jnp.int32(0)))

    pltpu.sync_copy(blo_v, lo_hbm.at[wid])
    pltpu.sync_copy(bhi_v, hi_hbm.at[wid])


def _to_sortable(x):
    top = jnp.uint32(0x80000000)
    bits = lax.bitcast_convert_type(x, jnp.uint32)
    return jnp.where(bits >= top, ~bits, bits | top)


def _from_sortable(u):
    top = jnp.uint32(0x80000000)
    bits = jnp.where(u >= top, u ^ top, ~u)
    return lax.bitcast_convert_type(bits, jnp.float32)


def _final_kernel(lo_ref, hi_ref, max_ref, min_ref, ulo_ref, uhi_ref):
    ulo_ref[...] = _to_sortable(lo_ref[...])
    uhi_ref[...] = _to_sortable(hi_ref[...])

    def body(i, carry):
        p_min, p_max = carry
        bit = jnp.left_shift(jnp.uint32(1), jnp.uint32(31) - i.astype(jnp.uint32))
        c_min = p_min | bit
        c_max = p_max | bit
        cnt_min = jnp.sum((ulo_ref[...] < c_min).astype(jnp.int32))
        cnt_max = jnp.sum((uhi_ref[...] < c_max).astype(jnp.int32))
        p_min = jnp.where(cnt_min >= _K_MIN, p_min, c_min)
        p_max = jnp.where(cnt_max >= _K_TOP, p_max, c_max)
        return p_min, p_max

    p_min, p_max = lax.fori_loop(0, 32, body, (jnp.uint32(0), jnp.uint32(0)))
    min_ref[0, 0] = _from_sortable(p_min)
    max_ref[0, 0] = _from_sortable(p_max)


@functools.cache
def _make_sc_filter():
    return functools.partial(
        pl.kernel,
        mesh=plsc.VectorSubcoreMesh(core_axis_name="c", subcore_axis_name="s"),
        compiler_params=pltpu.CompilerParams(needs_layout_passes=False),
        out_type=(
            jax.ShapeDtypeStruct((_NW, _CAP), jnp.float32),
            jax.ShapeDtypeStruct((_NW, _CAP), jnp.float32),
        ),
        scratch_types=[
            pltpu.VMEM((_CHUNK,), jnp.float32),
            pltpu.VMEM((_CAP,), jnp.float32),
            pltpu.VMEM((_CAP,), jnp.float32),
            pltpu.VMEM((_NBLK * _L,), jnp.int32),
            pltpu.SemaphoreType.DMA,
        ],
    )(_sc_filter_body)


def kernel(x):
    lo_c, hi_c = _make_sc_filter()(x)
    out_max, out_min = pl.pallas_call(
        _final_kernel,
        out_shape=(
            jax.ShapeDtypeStruct((1, 1), jnp.float32),
            jax.ShapeDtypeStruct((1, 1), jnp.float32),
        ),
        out_specs=(
            pl.BlockSpec(memory_space=pltpu.SMEM),
            pl.BlockSpec(memory_space=pltpu.SMEM),
        ),
        scratch_shapes=[
            pltpu.VMEM((_NW, _CAP), jnp.uint32),
            pltpu.VMEM((_NW, _CAP), jnp.uint32),
        ],
    )(lo_c, hi_c)
    return (out_max[0, 0], out_min[0, 0])
